# baseline (speedup 1.0000x reference)
import functools

import jax
import jax.numpy as jnp
from jax import lax
from jax.experimental import pallas as pl
from jax.experimental.pallas import tpu as pltpu
from jax.experimental.pallas import tpu_sc as plsc

EMB = 128
BATCH = 16384
TAB = 1000
TAB_PAD = 1024

NUM_CORES = 2
NUM_SUBCORES = 16
NW = NUM_CORES * NUM_SUBCORES

TC_B = 8192
SC_B = BATCH - TC_B
B_PER_W = SC_B // NW
CHUNK = 128
N_CHUNKS = B_PER_W // CHUNK

TC_BLK = 256


def _tc_body(idx_ref, table_ref, dst_ref, out_ref):
    del dst_ref
    ids = idx_ref[:]
    iota = lax.broadcasted_iota(jnp.int32, (TC_BLK, TAB_PAD), 1)
    onehot = (ids == iota).astype(jnp.float32)
    out_ref[...] = jnp.dot(onehot, table_ref[:],
                           preferred_element_type=jnp.float32)


def _tc_fill(idx_tc, table_pad, dst):
    return pl.pallas_call(
        _tc_body,
        grid=(TC_B // TC_BLK,),
        in_specs=[
            pl.BlockSpec((TC_BLK, 1), lambda i: (i, 0)),
            pl.BlockSpec((TAB_PAD, EMB), lambda i: (0, 0)),
            pl.BlockSpec(memory_space=pl.ANY),
        ],
        out_specs=pl.BlockSpec((TC_BLK, EMB), lambda i: (i, 0)),
        out_shape=jax.ShapeDtypeStruct((BATCH, EMB), jnp.float32),
        input_output_aliases={2: 0},
    )(idx_tc, table_pad, dst)


def _sc_half(idx_sc, table):
    mesh = plsc.VectorSubcoreMesh(core_axis_name="c", subcore_axis_name="s")

    @functools.partial(
        pl.kernel,
        mesh=mesh,
        out_type=jax.ShapeDtypeStruct((BATCH, EMB), jnp.float32),
        scratch_types=[
            pltpu.VMEM((N_CHUNKS, CHUNK), jnp.int32),
            pltpu.VMEM((N_CHUNKS, CHUNK, EMB), jnp.float32),
            pltpu.SemaphoreType.DMA,
        ],
    )
    def k(table_hbm, idx_hbm, out_hbm, idx_v, rows_v, sem):
        wid = lax.axis_index("s") * NUM_CORES + lax.axis_index("c")
        base = TC_B + wid * B_PER_W
        pltpu.sync_copy(idx_hbm.at[wid], idx_v)
        copies = [
            pltpu.async_copy(table_hbm.at[idx_v.at[j]], rows_v.at[j], sem)
            for j in range(N_CHUNKS)
        ]
        for c in copies:
            c.wait()
        stores = [
            pltpu.async_copy(
                rows_v.at[j], out_hbm.at[pl.ds(base + j * CHUNK, CHUNK)], sem
            )
            for j in range(N_CHUNKS)
        ]
        for s in stores:
            s.wait()

    return k(table, idx_sc)


@jax.jit
def _run(t, pos_encoding):
    idx = t.astype(jnp.int32)
    table_pad = jnp.pad(pos_encoding, ((0, TAB_PAD - TAB), (0, 0)))
    idx_tc = idx[:TC_B].reshape(TC_B, 1)
    idx_sc = idx[TC_B:].reshape(NW, N_CHUNKS, CHUNK)
    dst = _sc_half(idx_sc, pos_encoding)
    return _tc_fill(idx_tc, table_pad, dst)


def kernel(t, pos_encoding):
    return _run(t, pos_encoding)


# speedup vs baseline: 1.6190x; 1.6190x over previous
import functools

import jax
import jax.numpy as jnp
from jax import lax
from jax.experimental import pallas as pl
from jax.experimental.pallas import tpu as pltpu
from jax.experimental.pallas import tpu_sc as plsc

EMB = 128
BATCH = 16384
NUM_CORES = 2
NUM_SUBCORES = 16
NW = NUM_CORES * NUM_SUBCORES
B_PER_W = BATCH // NW
CHUNK = 128
N_CHUNKS = B_PER_W // CHUNK


@jax.jit
def _sc_gather(idx, table):
    mesh = plsc.VectorSubcoreMesh(core_axis_name="c", subcore_axis_name="s")

    @functools.partial(
        pl.kernel,
        mesh=mesh,
        out_type=jax.ShapeDtypeStruct((NW, N_CHUNKS, CHUNK, EMB), jnp.float32),
        scratch_types=[
            pltpu.VMEM((N_CHUNKS, CHUNK), jnp.int32),
            pltpu.VMEM((N_CHUNKS, CHUNK, EMB), jnp.float32),
        ]
        + [pltpu.SemaphoreType.DMA] * N_CHUNKS
        + [pltpu.SemaphoreType.DMA],
    )
    def k(table_hbm, idx_hbm, out_hbm, idx_v, rows_v, *sems):
        gsems, ssem = sems[:N_CHUNKS], sems[N_CHUNKS]
        wid = lax.axis_index("s") * NUM_CORES + lax.axis_index("c")
        out_w = out_hbm.at[wid]
        pltpu.sync_copy(idx_hbm.at[wid], idx_v)

        def gather(j):
            return pltpu.async_copy(
                table_hbm.at[idx_v.at[j]], rows_v.at[j], gsems[j]
            )

        gathers = [gather(0), gather(1)]
        stores = []
        for j in range(N_CHUNKS):
            gathers[j].wait()
            stores.append(pltpu.async_copy(rows_v.at[j], out_w.at[j], ssem))
            if j + 2 < N_CHUNKS:
                gathers.append(gather(j + 2))
        for s in stores:
            s.wait()

    return k(table, idx)


def kernel(t, pos_encoding):
    idx = t.astype(jnp.int32).reshape(NW, N_CHUNKS, CHUNK)
    out = _sc_gather(idx, pos_encoding)
    return out.reshape(BATCH, EMB)
